# Initial kernel scaffold; baseline (speedup 1.0000x reference)
import jax
import jax.numpy as jnp
from jax.experimental import pallas as pl


def kernel(points, planes):
    raise NotImplementedError("write your pallas kernel here")


# speedup vs baseline: 20.7215x; 20.7215x over previous
import functools

import jax
import jax.numpy as jnp
from jax.experimental import pallas as pl
from jax.experimental.pallas import tpu as pltpu

_THRESH = 0.05
_LANES = 128
_CHUNK_ROWS = 160


def _bf(v):
    return v.astype(jnp.bfloat16).astype(jnp.float32)


def _bf_host(v):
    u = jax.lax.bitcast_convert_type(v, jnp.uint32)
    lsb = (u >> 16) & jnp.uint32(1)
    u = (u + jnp.uint32(0x7FFF) + lsb) & jnp.uint32(0xFFFF0000)
    return jax.lax.bitcast_convert_type(u, jnp.float32)


def _padmask(chunk_idx, shape, num_points):
    rows = jax.lax.broadcasted_iota(jnp.int32, shape, 0)
    cols = jax.lax.broadcasted_iota(jnp.int32, shape, 1)
    gidx = (chunk_idx * _CHUNK_ROWS + rows) * _LANES + cols
    return jnp.where(gidx < num_points, 1.0, 0.0).astype(jnp.float32)


def _dist(xb, yb, zb, nx, ny, nz, d):
    return jnp.abs(xb * nx + yb * ny + zb * nz + d)


def _first_moments_kernel(num_points, num_planes, pts_ref, tab_ref, out_ref):
    c = pl.program_id(1)

    @pl.when(c == 0)
    def _init():
        out_ref[...] = jnp.zeros_like(out_ref)

    x = pts_ref[0]
    y = pts_ref[1]
    z = pts_ref[2]
    xb, yb, zb = _bf(x), _bf(y), _bf(z)
    pm = _padmask(c, x.shape, num_points)

    def plane_body(i, _):
        dist = _dist(xb, yb, zb, tab_ref[0, i], tab_ref[1, i], tab_ref[2, i],
                     tab_ref[3, i])
        m = jnp.where(dist < _THRESH, pm, 0.0)
        feats = (m, m * x, m * y, m * z)
        rowsum = jnp.concatenate(
            [jnp.sum(f, axis=0, keepdims=True) for f in feats]
            + [jnp.zeros((4, _LANES), jnp.float32)], axis=0)
        sl = pl.ds(i * 8, 8)
        out_ref[sl, :] = out_ref[sl, :] + rowsum
        return 0

    jax.lax.fori_loop(0, num_planes, plane_body, 0)


def _cov_kernel(num_points, num_planes, pts_ref, tab_ref, out_ref):
    c = pl.program_id(1)

    @pl.when(c == 0)
    def _init():
        out_ref[...] = jnp.zeros_like(out_ref)

    x = pts_ref[0]
    y = pts_ref[1]
    z = pts_ref[2]
    xb, yb, zb = _bf(x), _bf(y), _bf(z)
    pm = _padmask(c, x.shape, num_points)

    def plane_body(i, _):
        dist = _dist(xb, yb, zb, tab_ref[0, i], tab_ref[1, i], tab_ref[2, i],
                     tab_ref[3, i])
        m = jnp.where(dist < _THRESH, pm, 0.0)
        cx = _bf(m * (x - tab_ref[4, i]))
        cy = _bf(m * (y - tab_ref[5, i]))
        cz = _bf(m * (z - tab_ref[6, i]))
        feats = (cx * cx, cx * cy, cx * cz, cy * cy, cy * cz, cz * cz)
        rowsum = jnp.concatenate(
            [jnp.sum(f, axis=0, keepdims=True) for f in feats]
            + [jnp.zeros((2, _LANES), jnp.float32)], axis=0)
        sl = pl.ds(i * 8, 8)
        out_ref[sl, :] = out_ref[sl, :] + rowsum
        return 0

    jax.lax.fori_loop(0, num_planes, plane_body, 0)


def _project_kernel(num_planes, pts_ref, tab_ref, proj_ref, disp_ref):
    x = pts_ref[0]
    y = pts_ref[1]
    z = pts_ref[2]
    xb, yb, zb = _bf(x), _bf(y), _bf(z)

    def plane_body(i, carry):
        px, py, pz = carry
        dist = _dist(xb, yb, zb, tab_ref[0, i], tab_ref[1, i], tab_ref[2, i],
                     tab_ref[3, i])
        m = jnp.where(dist < _THRESH, tab_ref[12, i], 0.0)
        dot = (_bf(px) * tab_ref[4, i] + _bf(py) * tab_ref[5, i]
               + _bf(pz) * tab_ref[6, i] + tab_ref[7, i])
        t = dot * m
        return (px - tab_ref[8, i] * t, py - tab_ref[9, i] * t,
                pz - tab_ref[10, i] * t)

    px, py, pz = jax.lax.fori_loop(0, num_planes, plane_body, (x, y, z))
    proj_ref[0] = px
    proj_ref[1] = py
    proj_ref[2] = pz
    disp_ref[0] = px - x
    disp_ref[1] = py - y
    disp_ref[2] = pz - z


def kernel(points, planes):
    B, N, _ = points.shape
    M = planes.shape[1]
    f32 = jnp.float32
    points = points.astype(f32)
    planes = planes.astype(f32)

    raw = planes[..., :3]
    dvec = planes[..., 3]
    nrm = jnp.linalg.norm(raw, axis=-1)
    nhat = raw / jnp.maximum(nrm, 1e-12)[..., None]
    valid_normal = nrm >= 1e-6
    nhat_b = _bf_host(nhat)

    chunk_elems = _CHUNK_ROWS * _LANES
    num_chunks = -(-N // chunk_elems)
    R = num_chunks * _CHUNK_ROWS
    pad = R * _LANES - N
    pts_t = jnp.swapaxes(points, 1, 2)
    pts_p = jnp.pad(pts_t, ((0, 0), (0, 0), (0, pad))).reshape(B, 3, R,
                                                               _LANES)
    grid = (B, num_chunks)
    pts_spec = pl.BlockSpec((None, 3, _CHUNK_ROWS, _LANES),
                            lambda b, c: (b, 0, c, 0))

    def tab_spec(rows):
        return pl.BlockSpec((None, rows, M), lambda b, c: (b, 0, 0),
                            memory_space=pltpu.SMEM)

    acc_spec = pl.BlockSpec((None, 8 * M, _LANES), lambda b, c: (b, 0, 0))
    acc_shape = jax.ShapeDtypeStruct((B, 8 * M, _LANES), f32)

    tab_a = jnp.concatenate(
        [jnp.swapaxes(nhat_b, 1, 2), dvec[:, None, :],
         jnp.zeros((B, 4, M), f32)], axis=1)
    mom_part = pl.pallas_call(
        functools.partial(_first_moments_kernel, N, M),
        grid=grid,
        in_specs=[pts_spec, tab_spec(8)],
        out_specs=acc_spec,
        out_shape=acc_shape,
    )(pts_p, tab_a)
    mom = jnp.sum(mom_part, axis=-1).reshape(B, M, 8)
    cnt = mom[..., 0]
    s = mom[..., 1:4]
    safe_cnt = jnp.maximum(cnt, 1.0)
    cen = s / safe_cnt[..., None]
    valid = jnp.logical_and(valid_normal, cnt >= 3.0)

    tab_b = jnp.concatenate(
        [jnp.swapaxes(nhat_b, 1, 2), dvec[:, None, :],
         jnp.swapaxes(cen, 1, 2), jnp.zeros((B, 1, M), f32)], axis=1)
    cov_part = pl.pallas_call(
        functools.partial(_cov_kernel, N, M),
        grid=grid,
        in_specs=[pts_spec, tab_spec(8)],
        out_specs=acc_spec,
        out_shape=acc_shape,
    )(pts_p, tab_b)
    cv = jnp.sum(cov_part, axis=-1).reshape(B, M, 8)
    cov = jnp.stack([
        jnp.stack([cv[..., 0], cv[..., 1], cv[..., 2]], axis=-1),
        jnp.stack([cv[..., 1], cv[..., 3], cv[..., 4]], axis=-1),
        jnp.stack([cv[..., 2], cv[..., 4], cv[..., 5]], axis=-1),
    ], axis=-2)
    fallback = jnp.diag(jnp.array([3.0, 2.0, 1.0], f32))
    cov_safe = jnp.where(valid[..., None, None], cov, fallback)

    _, _, vh = jnp.linalg.svd(cov_safe)
    rn = vh[..., :, 2]
    flip = jnp.sum(rn * nhat, axis=-1) < 0
    rn = jnp.where(flip[..., None], -rn, rn)
    rd = -jnp.sum(cen * rn, axis=-1)

    tab_c = jnp.concatenate(
        [jnp.swapaxes(nhat_b, 1, 2), dvec[:, None, :],
         jnp.swapaxes(_bf_host(rn), 1, 2), rd[:, None, :],
         jnp.swapaxes(rn, 1, 2),
         jnp.zeros((B, 1, M), f32),
         valid.astype(f32)[:, None, :],
         jnp.zeros((B, 3, M), f32)], axis=1)
    proj_p, disp_p = pl.pallas_call(
        functools.partial(_project_kernel, M),
        grid=grid,
        in_specs=[pts_spec, tab_spec(16)],
        out_specs=[pts_spec, pts_spec],
        out_shape=[
            jax.ShapeDtypeStruct((B, 3, R, _LANES), f32),
            jax.ShapeDtypeStruct((B, 3, R, _LANES), f32),
        ],
    )(pts_p, tab_c)

    proj = jnp.swapaxes(proj_p.reshape(B, 3, R * _LANES)[:, :, :N], 1, 2)
    disp = jnp.swapaxes(disp_p.reshape(B, 3, R * _LANES)[:, :, :N], 1, 2)
    return proj, disp
